# baseline (speedup 1.0000x reference)
import functools

import jax
import jax.numpy as jnp
from jax import lax
from jax.experimental import pallas as pl
from jax.experimental.pallas import tpu as pltpu
from jax.experimental.pallas import tpu_sc as plsc

ROWS = 128
COLS = 32768
SC_ROWS = 32
TC_ROWS = ROWS - SC_ROWS
LANES = 16
NUM_WORKERS = 32
STREAMS = 8
SPAN = STREAMS * LANES
N_CHUNKS = 4
CHUNK = COLS // N_CHUNKS
CHUNK_STEPS = CHUNK // SPAN
INT_MAX = 2**31 - 1

TC_BLOCK = 2048
TC_STEPS = COLS // TC_BLOCK
SUB = 128
SUBS_PER_BLOCK = TC_BLOCK // SUB


@functools.partial(
    pl.kernel,
    out_type=jax.ShapeDtypeStruct((NUM_WORKERS, LANES), jnp.int32),
    mesh=plsc.VectorSubcoreMesh(core_axis_name="c", subcore_axis_name="s"),
    scratch_types=[
        pltpu.VMEM((COLS,), jnp.float32),
        pltpu.VMEM((LANES,), jnp.int32),
        pltpu.SemaphoreType.DMA,
    ],
)
def _argmax_sc(data_hbm, out_hbm, buf, res_ref, sem):
    cid = lax.axis_index("c")
    sid = lax.axis_index("s")
    wid = cid * 16 + sid
    lane = lax.iota(jnp.int32, LANES)

    handles = [
        pltpu.async_copy(
            data_hbm.at[wid, pl.ds(k * CHUNK, CHUNK)],
            buf.at[pl.ds(k * CHUNK, CHUNK)],
            sem,
        )
        for k in range(N_CHUNKS)
    ]

    neg_inf = jnp.full((LANES,), -jnp.inf, jnp.float32)
    zeros = jnp.zeros((LANES,), jnp.int32)
    carry = tuple([neg_inf] * STREAMS + [zeros] * STREAMS)

    def make_step(chunk_base_step):
        def step(t, carry):
            vals = carry[:STREAMS]
            steps = carry[STREAMS:]
            tt = chunk_base_step + t
            base = tt * SPAN
            new_vals, new_steps = [], []
            for s in range(STREAMS):
                v = buf[pl.ds(base + s * LANES, LANES)]
                c = v > vals[s]
                new_steps.append(jnp.where(c, tt, steps[s]))
                new_vals.append(jnp.maximum(vals[s], v))
            return tuple(new_vals + new_steps)
        return step

    for k in range(N_CHUNKS):
        handles[k].wait()
        carry = lax.fori_loop(0, CHUNK_STEPS, make_step(k * CHUNK_STEPS), carry)

    vals = carry[:STREAMS]
    steps = carry[STREAMS:]
    pairs = [
        (vals[s], steps[s] * SPAN + (s * LANES) + lane) for s in range(STREAMS)
    ]

    def merge(a, b):
        va, ia = a
        vb, ib = b
        take_b = (vb > va) | ((vb == va) & (ib < ia))
        return (jnp.where(take_b, vb, va), jnp.where(take_b, ib, ia))

    while len(pairs) > 1:
        pairs = [merge(pairs[i], pairs[i + 1]) for i in range(0, len(pairs), 2)]
    v, idx = pairs[0]

    for k in (8, 4, 2, 1):
        perm = lane ^ k
        vb = v.at[perm].get(mode="promise_in_bounds")
        ib = idx.at[perm].get(mode="promise_in_bounds")
        v, idx = merge((v, idx), (vb, ib))

    res_ref[...] = idx
    pltpu.sync_copy(res_ref, out_hbm.at[wid])


TC_BR = 32
TC_RBLKS = TC_ROWS // TC_BR


def _tc_body(x_ref, o_ref, vmax_ref, vcid_ref):
    step = pl.program_id(1)

    @pl.when(step == 0)
    def _():
        vmax_ref[...] = jnp.full((TC_BR, SUB), -jnp.inf, jnp.float32)
        vcid_ref[...] = jnp.zeros((TC_BR, SUB), jnp.int32)

    acc = vmax_ref[...]
    cid = vcid_ref[...]
    for k in range(SUBS_PER_BLOCK):
        xk = x_ref[:, k * SUB:(k + 1) * SUB]
        ck = step * SUBS_PER_BLOCK + k
        better = xk > acc
        cid = jnp.where(better, ck, cid)
        acc = jnp.where(better, xk, acc)
    vmax_ref[...] = acc
    vcid_ref[...] = cid

    @pl.when(step == TC_STEPS - 1)
    def _():
        rowmax = jnp.max(acc, axis=1, keepdims=True)
        gidx = cid * SUB + lax.broadcasted_iota(jnp.int32, (TC_BR, SUB), 1)
        cand = jnp.where(acc == rowmax, gidx, INT_MAX)
        o_ref[...] = jnp.min(cand, axis=1, keepdims=True)


_argmax_tc = pl.pallas_call(
    _tc_body,
    grid=(TC_RBLKS, TC_STEPS),
    in_specs=[
        pl.BlockSpec((TC_BR, TC_BLOCK), lambda r, i: (r + 1, i)),
    ],
    out_specs=pl.BlockSpec((TC_BR, 1), lambda r, i: (r, 0)),
    out_shape=jax.ShapeDtypeStruct((TC_ROWS, 1), jnp.int32),
    scratch_shapes=[
        pltpu.VMEM((TC_BR, SUB), jnp.float32),
        pltpu.VMEM((TC_BR, SUB), jnp.int32),
    ],
)


def kernel(data):
    sc2 = _argmax_sc(data)
    tc2 = _argmax_tc(data)
    return jnp.concatenate([sc2[:, 0], tc2[:, 0]])


# speedup vs baseline: 1.7490x; 1.7490x over previous
import functools

import jax
import jax.numpy as jnp
from jax import lax
from jax.experimental import pallas as pl
from jax.experimental.pallas import tpu as pltpu
from jax.experimental.pallas import tpu_sc as plsc

ROWS = 128
COLS = 32768
SC_ROWS = 32
TC_ROWS = ROWS - SC_ROWS
LANES = 16
NUM_WORKERS = 32
STREAMS = 8
SPAN = STREAMS * LANES
N_CHUNKS = 4
CHUNK = COLS // N_CHUNKS
CHUNK_STEPS = CHUNK // SPAN
INT_MAX = 2**31 - 1

TC_BLOCK = 8192
TC_STEPS = COLS // TC_BLOCK
SUB = 128
SUBS_PER_BLOCK = TC_BLOCK // SUB


@functools.partial(
    pl.kernel,
    out_type=jax.ShapeDtypeStruct((NUM_WORKERS, LANES), jnp.int32),
    mesh=plsc.VectorSubcoreMesh(core_axis_name="c", subcore_axis_name="s"),
    scratch_types=[
        pltpu.VMEM((COLS,), jnp.float32),
        pltpu.VMEM((LANES,), jnp.int32),
        pltpu.SemaphoreType.DMA,
    ],
)
def _argmax_sc(data_hbm, out_hbm, buf, res_ref, sem):
    cid = lax.axis_index("c")
    sid = lax.axis_index("s")
    wid = cid * 16 + sid
    row = TC_ROWS + wid
    lane = lax.iota(jnp.int32, LANES)

    handles = [
        pltpu.async_copy(
            data_hbm.at[row, pl.ds(k * CHUNK, CHUNK)],
            buf.at[pl.ds(k * CHUNK, CHUNK)],
            sem,
        )
        for k in range(N_CHUNKS)
    ]

    neg_inf = jnp.full((LANES,), -jnp.inf, jnp.float32)
    zeros = jnp.zeros((LANES,), jnp.int32)
    carry = tuple([neg_inf] * STREAMS + [zeros] * STREAMS)

    def make_step(chunk_base_step):
        def step(t, carry):
            vals = carry[:STREAMS]
            steps = carry[STREAMS:]
            tt = chunk_base_step + t
            base = tt * SPAN
            new_vals, new_steps = [], []
            for s in range(STREAMS):
                v = buf[pl.ds(base + s * LANES, LANES)]
                c = v > vals[s]
                new_steps.append(jnp.where(c, tt, steps[s]))
                new_vals.append(jnp.maximum(vals[s], v))
            return tuple(new_vals + new_steps)
        return step

    for k in range(N_CHUNKS):
        handles[k].wait()
        carry = lax.fori_loop(0, CHUNK_STEPS, make_step(k * CHUNK_STEPS), carry)

    vals = carry[:STREAMS]
    steps = carry[STREAMS:]
    pairs = [
        (vals[s], steps[s] * SPAN + (s * LANES) + lane) for s in range(STREAMS)
    ]

    def merge(a, b):
        va, ia = a
        vb, ib = b
        take_b = (vb > va) | ((vb == va) & (ib < ia))
        return (jnp.where(take_b, vb, va), jnp.where(take_b, ib, ia))

    while len(pairs) > 1:
        pairs = [merge(pairs[i], pairs[i + 1]) for i in range(0, len(pairs), 2)]
    v, idx = pairs[0]

    for k in (8, 4, 2, 1):
        perm = lane ^ k
        vb = v.at[perm].get(mode="promise_in_bounds")
        ib = idx.at[perm].get(mode="promise_in_bounds")
        v, idx = merge((v, idx), (vb, ib))

    res_ref[...] = idx
    pltpu.sync_copy(res_ref, out_hbm.at[wid])


TC_BR = 32
TC_RBLKS = TC_ROWS // TC_BR


def _tc_body(x_ref, o_ref, vmax_ref, vcid_ref):
    step = pl.program_id(0)

    @pl.when(step == 0)
    def _():
        vmax_ref[...] = jnp.full((TC_ROWS, SUB), -jnp.inf, jnp.float32)
        vcid_ref[...] = jnp.zeros((TC_ROWS, SUB), jnp.int32)

    acc = vmax_ref[...]
    cid = vcid_ref[...]
    for k in range(SUBS_PER_BLOCK):
        xk = x_ref[:, k * SUB:(k + 1) * SUB]
        ck = step * SUBS_PER_BLOCK + k
        better = xk > acc
        cid = jnp.where(better, ck, cid)
        acc = jnp.where(better, xk, acc)
    vmax_ref[...] = acc
    vcid_ref[...] = cid

    @pl.when(step == TC_STEPS - 1)
    def _():
        rowmax = jnp.max(acc, axis=1, keepdims=True)
        gidx = cid * SUB + lax.broadcasted_iota(jnp.int32, (TC_ROWS, SUB), 1)
        cand = jnp.where(acc == rowmax, gidx, INT_MAX)
        o_ref[...] = jnp.min(cand, axis=1, keepdims=True)


_argmax_tc = pl.pallas_call(
    _tc_body,
    grid=(TC_STEPS,),
    in_specs=[
        pl.BlockSpec((TC_ROWS, TC_BLOCK), lambda i: (0, i)),
    ],
    out_specs=pl.BlockSpec((TC_ROWS, 1), lambda i: (0, 0)),
    out_shape=jax.ShapeDtypeStruct((TC_ROWS, 1), jnp.int32),
    scratch_shapes=[
        pltpu.VMEM((TC_ROWS, SUB), jnp.float32),
        pltpu.VMEM((TC_ROWS, SUB), jnp.int32),
    ],
)


def kernel(data):
    sc2 = _argmax_sc(data)
    tc2 = _argmax_tc(data)
    return jnp.concatenate([tc2[:, 0], sc2[:, 0]])


# speedup vs baseline: 1.7558x; 1.0039x over previous
import functools

import jax
import jax.numpy as jnp
from jax import lax
from jax.experimental import pallas as pl
from jax.experimental.pallas import tpu as pltpu
from jax.experimental.pallas import tpu_sc as plsc

ROWS = 128
COLS = 32768
SC_ROWS = 32
TC_ROWS = ROWS - SC_ROWS
LANES = 16
NUM_WORKERS = 32
STREAMS = 8
SPAN = STREAMS * LANES
N_CHUNKS = 4
CHUNK = COLS // N_CHUNKS
CHUNK_STEPS = CHUNK // SPAN
INT_MAX = 2**31 - 1

TC_BLOCK = 8192
TC_STEPS = COLS // TC_BLOCK
SUB = 128
SUBS_PER_BLOCK = TC_BLOCK // SUB


@functools.partial(
    pl.kernel,
    out_type=jax.ShapeDtypeStruct((NUM_WORKERS, LANES), jnp.int32),
    mesh=plsc.VectorSubcoreMesh(core_axis_name="c", subcore_axis_name="s"),
    scratch_types=[
        pltpu.VMEM((COLS,), jnp.float32),
        pltpu.VMEM((LANES,), jnp.int32),
        pltpu.SemaphoreType.DMA,
    ],
)
def _argmax_sc(data_hbm, out_hbm, buf, res_ref, sem):
    cid = lax.axis_index("c")
    sid = lax.axis_index("s")
    wid = cid * 16 + sid
    row = TC_ROWS + wid
    lane = lax.iota(jnp.int32, LANES)

    handles = [
        pltpu.async_copy(
            data_hbm.at[row, pl.ds(k * CHUNK, CHUNK)],
            buf.at[pl.ds(k * CHUNK, CHUNK)],
            sem,
        )
        for k in range(N_CHUNKS)
    ]

    neg_inf = jnp.full((LANES,), -jnp.inf, jnp.float32)
    zeros = jnp.zeros((LANES,), jnp.int32)
    carry = tuple([neg_inf] * STREAMS + [zeros] * STREAMS)

    def make_step(chunk_base_step):
        def step(t, carry):
            vals = carry[:STREAMS]
            steps = carry[STREAMS:]
            tt = chunk_base_step + t
            base = tt * SPAN
            new_vals, new_steps = [], []
            for s in range(STREAMS):
                v = buf[pl.ds(base + s * LANES, LANES)]
                c = v > vals[s]
                new_steps.append(jnp.where(c, tt, steps[s]))
                new_vals.append(jnp.maximum(vals[s], v))
            return tuple(new_vals + new_steps)
        return step

    for k in range(N_CHUNKS):
        handles[k].wait()
        carry = lax.fori_loop(0, CHUNK_STEPS, make_step(k * CHUNK_STEPS), carry)

    vals = carry[:STREAMS]
    steps = carry[STREAMS:]
    pairs = [
        (vals[s], steps[s] * SPAN + (s * LANES) + lane) for s in range(STREAMS)
    ]

    def merge(a, b):
        va, ia = a
        vb, ib = b
        take_b = (vb > va) | ((vb == va) & (ib < ia))
        return (jnp.where(take_b, vb, va), jnp.where(take_b, ib, ia))

    while len(pairs) > 1:
        pairs = [merge(pairs[i], pairs[i + 1]) for i in range(0, len(pairs), 2)]
    v, idx = pairs[0]

    for k in (8, 4, 2, 1):
        perm = lane ^ k
        vb = v.at[perm].get(mode="promise_in_bounds")
        ib = idx.at[perm].get(mode="promise_in_bounds")
        v, idx = merge((v, idx), (vb, ib))

    res_ref[...] = idx
    pltpu.sync_copy(res_ref, out_hbm.at[wid])


TC_BR = 32
TC_RBLKS = TC_ROWS // TC_BR


def _tc_body(x_ref, o_ref, vmax_ref, vcid_ref):
    step = pl.program_id(0)

    @pl.when(step == 0)
    def _():
        vmax_ref[...] = jnp.full((TC_ROWS, SUB), -jnp.inf, jnp.float32)
        vcid_ref[...] = jnp.zeros((TC_ROWS, SUB), jnp.int32)

    acc = vmax_ref[...]
    cid = vcid_ref[...]
    for k in range(SUBS_PER_BLOCK):
        xk = x_ref[:, k * SUB:(k + 1) * SUB]
        ck = step * SUBS_PER_BLOCK + k
        better = xk > acc
        cid = jnp.where(better, ck, cid)
        acc = jnp.where(better, xk, acc)
    vmax_ref[...] = acc
    vcid_ref[...] = cid

    @pl.when(step == TC_STEPS - 1)
    def _():
        rowmax = jnp.max(acc, axis=1, keepdims=True)
        gidx = cid * SUB + lax.broadcasted_iota(jnp.int32, (TC_ROWS, SUB), 1)
        cand = jnp.where(acc == rowmax, gidx, INT_MAX)
        o_ref[...] = jnp.min(cand, axis=1, keepdims=True)


_argmax_tc = pl.pallas_call(
    _tc_body,
    grid=(TC_STEPS,),
    in_specs=[
        pl.BlockSpec((TC_ROWS, TC_BLOCK), lambda i: (0, i)),
    ],
    out_specs=pl.BlockSpec((TC_ROWS, 1), lambda i: (0, 0)),
    out_shape=jax.ShapeDtypeStruct((TC_ROWS, 1), jnp.int32),
    scratch_shapes=[
        pltpu.VMEM((TC_ROWS, SUB), jnp.float32),
        pltpu.VMEM((TC_ROWS, SUB), jnp.int32),
    ],
)


def kernel(data):
    sc2 = _argmax_sc(data)
    tc2 = _argmax_tc(data)
    return jnp.concatenate([tc2, sc2[:, :1]], axis=0)[:, 0]
